# baseline (speedup 1.0000x reference)
import functools

import jax
import jax.numpy as jnp
from jax import lax
from jax.experimental import pallas as pl
from jax.experimental.pallas import tpu as pltpu
from jax.experimental.pallas import tpu_sc as plsc

_NC = 2
_NS = 16
_NW = _NC * _NS
_L = 16


def _build_sc_lookup(b_pad: int):
    b_per_w = b_pad // _NW
    nchunk = b_per_w // _L
    mesh = plsc.VectorSubcoreMesh(core_axis_name="c", subcore_axis_name="s")

    @functools.partial(
        pl.kernel,
        mesh=mesh,
        out_type=jax.ShapeDtypeStruct((b_pad,), jnp.float32),
        scratch_types=[
            pltpu.VMEM((b_per_w,), jnp.int32),
            pltpu.VMEM((4 * _L,), jnp.float32),
            pltpu.VMEM((b_per_w,), jnp.float32),
        ],
    )
    def sc_lookup(idx_hbm, table_hbm, out_hbm, idx_v, table_v, out_v):
        wid = lax.axis_index("s") * _NC + lax.axis_index("c")
        base = wid * b_per_w
        pltpu.sync_copy(table_hbm, table_v)
        pltpu.sync_copy(idx_hbm.at[pl.ds(base, b_per_w)], idx_v)

        t0 = table_v[pl.ds(0, _L)]
        t1 = table_v[pl.ds(_L, _L)]
        t2 = table_v[pl.ds(2 * _L, _L)]
        t3 = table_v[pl.ds(3 * _L, _L)]

        def body(i, carry):
            idx = idx_v[pl.ds(i * _L, _L)]
            lo = jnp.where(idx == 0, t0, t1)
            hi = jnp.where(idx == 2, t2, t3)
            out_v[pl.ds(i * _L, _L)] = jnp.where(idx < 2, lo, hi)
            return carry

        lax.fori_loop(0, nchunk, body, 0)
        pltpu.sync_copy(out_v, out_hbm.at[pl.ds(base, b_per_w)])

    return sc_lookup


def kernel(atom_types, scatter_norm_factor):
    n = atom_types.shape[0]
    grain = _NW * _L
    b_pad = ((n + grain - 1) // grain) * grain
    idx = jnp.zeros((b_pad,), jnp.int32).at[:n].set(atom_types)
    table = jnp.repeat(scatter_norm_factor[:, 0], _L)
    out = _build_sc_lookup(b_pad)(idx, table)
    return out[:n].reshape(n, 1)


# speedup vs baseline: 1.1056x; 1.1056x over previous
import functools

import jax
import jax.numpy as jnp
from jax import lax
from jax.experimental import pallas as pl
from jax.experimental.pallas import tpu as pltpu
from jax.experimental.pallas import tpu_sc as plsc

_NC = 2
_NS = 16
_NW = _NC * _NS
_L = 16
_UNROLL = 4


def _lane_gather(table_vec, idx):
    return lax.gather(
        table_vec,
        idx[:, None],
        lax.GatherDimensionNumbers(
            offset_dims=(), collapsed_slice_dims=(0,), start_index_map=(0,)
        ),
        (1,),
        mode=lax.GatherScatterMode.PROMISE_IN_BOUNDS,
    )


def _build_sc_lookup(n: int, t: int):
    chunk = ((n + _NW - 1) // _NW + 7) // 8 * 8
    last = n - (_NW - 1) * chunk
    assert 0 < last <= chunk
    nchunk = (chunk + _L - 1) // _L
    nchunk = (nchunk + _UNROLL - 1) // _UNROLL * _UNROLL
    buf = nchunk * _L
    mesh = plsc.VectorSubcoreMesh(core_axis_name="c", subcore_axis_name="s")

    @functools.partial(
        pl.kernel,
        mesh=mesh,
        out_type=jax.ShapeDtypeStruct((n,), jnp.float32),
        scratch_types=[
            pltpu.VMEM((buf,), jnp.int32),
            pltpu.VMEM((_L,), jnp.float32),
            pltpu.VMEM((buf,), jnp.float32),
        ],
    )
    def sc_lookup(idx_hbm, table_hbm, out_hbm, idx_v, table_v, out_v):
        wid = lax.axis_index("s") * _NC + lax.axis_index("c")
        base = wid * chunk
        pltpu.sync_copy(table_hbm, table_v.at[pl.ds(0, t)])

        is_last = wid == _NW - 1

        @pl.when(jnp.logical_not(is_last))
        def _():
            pltpu.sync_copy(idx_hbm.at[pl.ds(base, chunk)], idx_v.at[pl.ds(0, chunk)])

        @pl.when(is_last)
        def _():
            pltpu.sync_copy(idx_hbm.at[pl.ds(base, last)], idx_v.at[pl.ds(0, last)])

        tv = table_v[...]

        def body(i, carry):
            for u in range(_UNROLL):
                off = (i * _UNROLL + u) * _L
                out_v[pl.ds(off, _L)] = _lane_gather(tv, idx_v[pl.ds(off, _L)])
            return carry

        lax.fori_loop(0, nchunk // _UNROLL, body, 0)

        @pl.when(jnp.logical_not(is_last))
        def _():
            pltpu.sync_copy(out_v.at[pl.ds(0, chunk)], out_hbm.at[pl.ds(base, chunk)])

        @pl.when(is_last)
        def _():
            pltpu.sync_copy(out_v.at[pl.ds(0, last)], out_hbm.at[pl.ds(base, last)])

    return sc_lookup


def kernel(atom_types, scatter_norm_factor):
    n = atom_types.shape[0]
    t = scatter_norm_factor.shape[0]
    table = scatter_norm_factor.reshape(t)
    out = _build_sc_lookup(n, t)(atom_types, table)
    return out.reshape(n, 1)


# speedup vs baseline: 1.1189x; 1.0120x over previous
import functools

import jax
import jax.numpy as jnp
from jax import lax
from jax.experimental import pallas as pl
from jax.experimental.pallas import tpu as pltpu
from jax.experimental.pallas import tpu_sc as plsc

_NC = 2
_NS = 16
_NW = _NC * _NS
_L = 16
_UNROLL = 8


def _lane_gather(table_vec, idx):
    return lax.gather(
        table_vec,
        idx[:, None],
        lax.GatherDimensionNumbers(
            offset_dims=(), collapsed_slice_dims=(0,), start_index_map=(0,)
        ),
        (1,),
        mode=lax.GatherScatterMode.PROMISE_IN_BOUNDS,
    )


def _build_sc_lookup(n: int, t: int):
    chunk = ((n + _NW - 1) // _NW + 7) // 8 * 8
    last = n - (_NW - 1) * chunk
    assert 0 < last <= chunk
    nchunk = (chunk + _L - 1) // _L
    nchunk = (nchunk + _UNROLL - 1) // _UNROLL * _UNROLL
    buf = nchunk * _L
    mesh = plsc.VectorSubcoreMesh(core_axis_name="c", subcore_axis_name="s")

    @functools.partial(
        pl.kernel,
        mesh=mesh,
        out_type=jax.ShapeDtypeStruct((n,), jnp.float32),
        scratch_types=[
            pltpu.VMEM((buf,), jnp.int32),
            pltpu.VMEM((_L,), jnp.float32),
            pltpu.VMEM((buf,), jnp.float32),
            pltpu.SemaphoreType.DMA,
            pltpu.SemaphoreType.DMA,
        ],
    )
    def sc_lookup(idx_hbm, table_hbm, out_hbm, idx_v, table_v, out_v, sem0, sem1):
        wid = lax.axis_index("s") * _NC + lax.axis_index("c")
        base = wid * chunk
        cp_t = pltpu.async_copy(table_hbm, table_v.at[pl.ds(0, t)], sem0)

        is_last = wid == _NW - 1

        @pl.when(jnp.logical_not(is_last))
        def _():
            pltpu.async_copy(
                idx_hbm.at[pl.ds(base, chunk)], idx_v.at[pl.ds(0, chunk)], sem1
            ).wait()

        @pl.when(is_last)
        def _():
            pltpu.async_copy(
                idx_hbm.at[pl.ds(base, last)], idx_v.at[pl.ds(0, last)], sem1
            ).wait()

        cp_t.wait()
        tv = table_v[...]

        def body(i, carry):
            for u in range(_UNROLL):
                off = (i * _UNROLL + u) * _L
                out_v[pl.ds(off, _L)] = _lane_gather(tv, idx_v[pl.ds(off, _L)])
            return carry

        lax.fori_loop(0, nchunk // _UNROLL, body, 0)

        @pl.when(jnp.logical_not(is_last))
        def _():
            pltpu.sync_copy(
                out_v.at[pl.ds(0, chunk)], out_hbm.at[pl.ds(base, chunk)]
            )

        @pl.when(is_last)
        def _():
            pltpu.sync_copy(
                out_v.at[pl.ds(0, last)], out_hbm.at[pl.ds(base, last)]
            )

    return sc_lookup


def kernel(atom_types, scatter_norm_factor):
    n = atom_types.shape[0]
    t = scatter_norm_factor.shape[0]
    table = scatter_norm_factor.reshape(t)
    out = _build_sc_lookup(n, t)(atom_types, table)
    return out.reshape(n, 1)
